# Initial kernel scaffold; baseline (speedup 1.0000x reference)
import jax
import jax.numpy as jnp
from jax.experimental import pallas as pl


def kernel(features, edge_index, W1, b1, W2, b2, W3, b3):
    raise NotImplementedError("write your pallas kernel here")


# speedup vs baseline: 2.7154x; 2.7154x over previous
import functools

import jax
import jax.numpy as jnp
from jax import lax
from jax.experimental import pallas as pl
from jax.experimental.pallas import tpu as pltpu
from jax.experimental.pallas import tpu_sc as plsc

_NC = 2
_NS = 16
_CH = 128


def _n_pad(n_nodes):
    return ((n_nodes + 128) // 128) * 128


def _make_degrees(n_nodes, n_chunks):
    n_pad = _n_pad(n_nodes)
    rpt = n_pad // _NS
    mesh = plsc.VectorSubcoreMesh(core_axis_name="c", subcore_axis_name="s")

    @functools.partial(
        pl.kernel,
        out_type=(jax.ShapeDtypeStruct((_NC, n_pad, 128), jnp.float32),
                  jax.ShapeDtypeStruct((_NC, n_pad, 128), jnp.float32)),
        mesh=mesh,
        scratch_types=[
            pltpu.VMEM((n_chunks, _CH), jnp.int32),
            pltpu.VMEM((n_chunks, _CH), jnp.int32),
            pltpu.VMEM((_CH, 128), jnp.float32),
            pltpu.VMEM_SHARED((n_pad, 128), jnp.float32),
        ],
    )
    def k(src2d, dst2d, ones_hbm, zeros_hbm, out_o, out_i,
          sidx, didx, ones_v, hist):
        c = lax.axis_index("c")
        s = lax.axis_index("s")
        w = c * _NS + s
        pltpu.sync_copy(ones_hbm, ones_v)
        pltpu.sync_copy(src2d.at[w], sidx)
        pltpu.sync_copy(dst2d.at[w], didx)

        for idx, out in ((sidx, out_o), (didx, out_i)):
            pltpu.sync_copy(zeros_hbm.at[pl.ds(s * rpt, rpt)],
                            hist.at[pl.ds(s * rpt, rpt)])
            plsc.subcore_barrier()

            def body(i, carry, idx=idx):
                pltpu.sync_copy(ones_v, hist.at[idx.at[i]], add=True)
                return carry

            lax.fori_loop(0, n_chunks, body, 0)
            plsc.subcore_barrier()
            pltpu.sync_copy(hist.at[pl.ds(s * rpt, rpt)],
                            out.at[c, pl.ds(s * rpt, rpt)])
            plsc.subcore_barrier()

    return k


_BLK = 4


def _agg_edge_loop_blocked(table, agg, src4, dst4, s, sidx, didx,
                           buf0, buf1, sems, n_blk):
    sem_g0, sem_g1, sem_is, sem_id = sems
    pltpu.sync_copy(src4.at[s, 0], sidx.at[0])
    pltpu.sync_copy(dst4.at[s, 0], didx.at[0])
    pltpu.async_copy(src4.at[s, 1], sidx.at[1], sem_is)
    pltpu.async_copy(dst4.at[s, 1], didx.at[1], sem_id)
    pltpu.async_copy(table.at[sidx.at[0, 0]], buf0, sem_g0)

    def block(b, p):
        q = 1 - p
        for c in range(_BLK):
            buf, sem = (buf0, sem_g0) if c % 2 == 0 else (buf1, sem_g1)
            nbuf, nsem = (buf1, sem_g1) if c % 2 == 0 else (buf0, sem_g0)
            pltpu.make_async_copy(table.at[sidx.at[p, c]], buf, sem).wait()
            if c + 1 < _BLK:
                pltpu.async_copy(table.at[sidx.at[p, c + 1]], nbuf, nsem)
            else:
                @pl.when(b + 1 < n_blk)
                def _():
                    pltpu.make_async_copy(src4.at[s, b + 1], sidx.at[q],
                                          sem_is).wait()
                    pltpu.make_async_copy(dst4.at[s, b + 1], didx.at[q],
                                          sem_id).wait()
                    pltpu.async_copy(table.at[sidx.at[q, 0]], nbuf, nsem)
            pltpu.sync_copy(buf, agg.at[didx.at[p, c]], add=True)
        @pl.when(b + 2 < n_blk)
        def _():
            pltpu.async_copy(src4.at[s, b + 2], sidx.at[p], sem_is)
            pltpu.async_copy(dst4.at[s, b + 2], didx.at[p], sem_id)

    def body(j, carry):
        block(2 * j, 0)
        block(2 * j + 1, 1)
        return carry

    lax.fori_loop(0, n_blk // 2, body, 0)


def _make_agg_fsplit(n_nodes, d_half, n_chunks):
    n_pad = _n_pad(n_nodes)
    rpt = n_pad // _NS
    n_blk = n_chunks // _BLK
    mesh = plsc.VectorSubcoreMesh(core_axis_name="c", subcore_axis_name="s")

    @functools.partial(
        pl.kernel,
        out_type=jax.ShapeDtypeStruct((_NC, n_pad, d_half), jnp.float32),
        mesh=mesh,
        scratch_types=[
            pltpu.VMEM((2, _BLK, _CH), jnp.int32),
            pltpu.VMEM((2, _BLK, _CH), jnp.int32),
            pltpu.VMEM((_CH, d_half), jnp.float32),
            pltpu.VMEM((_CH, d_half), jnp.float32),
            pltpu.VMEM_SHARED((n_pad, d_half), jnp.float32),
            pltpu.SemaphoreType.DMA,
            pltpu.SemaphoreType.DMA,
            pltpu.SemaphoreType.DMA,
            pltpu.SemaphoreType.DMA,
        ],
    )
    def k(h0, h1, src4, dst4, zeros_hbm, out,
          sidx, didx, buf0, buf1, agg, sem0, sem1, sem_is, sem_id):
        c = lax.axis_index("c")
        s = lax.axis_index("s")
        pltpu.sync_copy(zeros_hbm.at[pl.ds(s * rpt, rpt)],
                        agg.at[pl.ds(s * rpt, rpt)])
        plsc.subcore_barrier()
        sems = (sem0, sem1, sem_is, sem_id)

        @pl.when(c == 0)
        def _():
            _agg_edge_loop_blocked(h0, agg, src4, dst4, s, sidx, didx,
                                   buf0, buf1, sems, n_blk)

        @pl.when(c == 1)
        def _():
            _agg_edge_loop_blocked(h1, agg, src4, dst4, s, sidx, didx,
                                   buf0, buf1, sems, n_blk)

        plsc.subcore_barrier()
        pltpu.sync_copy(agg.at[pl.ds(s * rpt, rpt)],
                        out.at[c, pl.ds(s * rpt, rpt)])

    return k


def _make_agg_esplit(n_nodes, n_chunks):
    n_pad = _n_pad(n_nodes)
    rpt = n_pad // _NS
    n_blk = n_chunks // _BLK
    mesh = plsc.VectorSubcoreMesh(core_axis_name="c", subcore_axis_name="s")

    @functools.partial(
        pl.kernel,
        out_type=jax.ShapeDtypeStruct((_NC, n_pad, 128), jnp.float32),
        mesh=mesh,
        scratch_types=[
            pltpu.VMEM((2, _BLK, _CH), jnp.int32),
            pltpu.VMEM((2, _BLK, _CH), jnp.int32),
            pltpu.VMEM((_CH, 128), jnp.float32),
            pltpu.VMEM((_CH, 128), jnp.float32),
            pltpu.VMEM_SHARED((n_pad, 128), jnp.float32),
            pltpu.SemaphoreType.DMA,
            pltpu.SemaphoreType.DMA,
            pltpu.SemaphoreType.DMA,
            pltpu.SemaphoreType.DMA,
        ],
    )
    def k(h, src4, dst4, zeros_hbm, out,
          sidx, didx, buf0, buf1, agg, sem0, sem1, sem_is, sem_id):
        c = lax.axis_index("c")
        s = lax.axis_index("s")
        w = c * _NS + s
        pltpu.sync_copy(zeros_hbm.at[pl.ds(s * rpt, rpt)],
                        agg.at[pl.ds(s * rpt, rpt)])
        plsc.subcore_barrier()
        _agg_edge_loop_blocked(h, agg, src4, dst4, w, sidx, didx,
                               buf0, buf1, (sem0, sem1, sem_is, sem_id),
                               n_blk)
        plsc.subcore_barrier()
        pltpu.sync_copy(agg.at[pl.ds(s * rpt, rpt)],
                        out.at[c, pl.ds(s * rpt, rpt)])

    return k


def _norm_from_deg(deg_ref):
    d = deg_ref[0, :, 0:1] + deg_ref[1, :, 0:1]
    return jnp.where(d > 0, lax.rsqrt(jnp.maximum(d, 1.0)), 0.0)


def _b1_body(x_ref, w_ref, dego_ref, out_ref):
    norm = _norm_from_deg(dego_ref)
    h = jnp.dot(x_ref[...], w_ref[...], preferred_element_type=jnp.float32)
    out_ref[0] = h * norm


def _b2_body(agg_ref, dego_ref, degi_ref, b1_ref, w2_ref, out_ref):
    norm_s = _norm_from_deg(dego_ref)
    norm_d = _norm_from_deg(degi_ref)
    x0 = jnp.maximum(agg_ref[0] * norm_d + b1_ref[:, 0:128], 0.0)
    x1 = jnp.maximum(agg_ref[1] * norm_d + b1_ref[:, 128:256], 0.0)
    h = (jnp.dot(x0, w2_ref[0:128, :], preferred_element_type=jnp.float32)
         + jnp.dot(x1, w2_ref[128:256, :], preferred_element_type=jnp.float32))
    hp = jnp.pad(h * norm_s, ((0, 0), (0, 128 - h.shape[1])))
    out_ref[...] = hp


def _b3_body(aggp_ref, dego_ref, degi_ref, b2_ref, w3_ref, out_ref):
    norm_s = _norm_from_deg(dego_ref)
    norm_d = _norm_from_deg(degi_ref)
    agg = aggp_ref[0, :, 0:32] + aggp_ref[1, :, 0:32]
    x = jnp.maximum(agg * norm_d + b2_ref[...], 0.0)
    h = jnp.dot(x, w3_ref[...], preferred_element_type=jnp.float32)
    hp = jnp.pad(h * norm_s, ((0, 0), (0, 128 - h.shape[1])))
    out_ref[...] = hp


def _b4_body(aggp_ref, degi_ref, b3_ref, out_ref):
    norm_d = _norm_from_deg(degi_ref)
    x = (aggp_ref[0, :, 0:16] + aggp_ref[1, :, 0:16]) * norm_d + b3_ref[...]
    cols = lax.broadcasted_iota(jnp.int32, x.shape, 1)
    valid = cols < 7
    xm = jnp.where(valid, x, -1e30)
    m = jnp.max(xm, axis=1, keepdims=True)
    ssum = jnp.sum(jnp.where(valid, jnp.exp(x - m), 0.0), axis=1,
                   keepdims=True)
    out_ref[...] = (x - m - jnp.log(ssum))[:, 0:7]


def kernel(features, edge_index, W1, b1, W2, b2, W3, b3):
    n, d_in = features.shape
    d_h1 = W1.shape[1]
    d_h2 = W2.shape[1]
    d_out = W3.shape[1]
    e = edge_index.shape[1]
    assert d_h1 == 256 and d_h2 == 32 and d_out == 7
    n_pad = _n_pad(n)

    epw = _NC * _NS * _CH
    e_pad = ((e + epw - 1) // epw) * epw
    sent = jnp.full((e_pad - e,), n, dtype=jnp.int32)
    src = jnp.concatenate([edge_index[0].astype(jnp.int32), sent])
    dst = jnp.concatenate([edge_index[1].astype(jnp.int32), sent])
    nch32 = e_pad // (_NC * _NS * _CH)
    nch16 = e_pad // (_NS * _CH)
    src32 = src.reshape(_NC * _NS, nch32 // _BLK, _BLK, _CH)
    dst32 = dst.reshape(_NC * _NS, nch32 // _BLK, _BLK, _CH)
    src16 = src.reshape(_NS, nch16 // _BLK, _BLK, _CH)
    dst16 = dst.reshape(_NS, nch16 // _BLK, _BLK, _CH)

    zeros128 = jnp.zeros((n_pad, 128), jnp.float32)
    b1r = b1.reshape(1, d_h1)
    b2r = b2.reshape(1, d_h2)
    w3p = jnp.pad(W3, ((0, 0), (0, 16 - d_out)))
    b3p = jnp.pad(b3, (0, 16 - d_out)).reshape(1, 16)

    srcd = src.reshape(_NC * _NS, nch32, _CH)
    dstd = dst.reshape(_NC * _NS, nch32, _CH)
    ones128 = jnp.ones((_CH, 128), jnp.float32)
    deg_o, deg_i = _make_degrees(n, nch32)(srcd, dstd, ones128, zeros128)

    rb = 1000
    gr = n // rb
    h1n = pl.pallas_call(
        _b1_body,
        grid=(gr, 2),
        in_specs=[
            pl.BlockSpec((rb, d_in), lambda i, j: (i, 0)),
            pl.BlockSpec((d_in, 128), lambda i, j: (0, j)),
            pl.BlockSpec((2, rb, 128), lambda i, j: (0, i, 0)),
        ],
        out_specs=pl.BlockSpec((1, rb, 128), lambda i, j: (j, i, 0)),
        out_shape=jax.ShapeDtypeStruct((2, n_pad, 128), jnp.float32),
    )(features, W1, deg_o)

    agg1 = _make_agg_fsplit(n, 128, nch16)(
        h1n[0], h1n[1], src16, dst16, zeros128)

    h2n = pl.pallas_call(
        _b2_body,
        grid=(gr,),
        in_specs=[
            pl.BlockSpec((2, rb, 128), lambda i: (0, i, 0)),
            pl.BlockSpec((2, rb, 128), lambda i: (0, i, 0)),
            pl.BlockSpec((2, rb, 128), lambda i: (0, i, 0)),
            pl.BlockSpec((1, d_h1), lambda i: (0, 0)),
            pl.BlockSpec((d_h1, d_h2), lambda i: (0, 0)),
        ],
        out_specs=pl.BlockSpec((rb, 128), lambda i: (i, 0)),
        out_shape=jax.ShapeDtypeStruct((n_pad, 128), jnp.float32),
    )(agg1, deg_o, deg_i, b1r, W2)

    agg2 = _make_agg_esplit(n, nch32)(h2n, src32, dst32, zeros128)

    h3n = pl.pallas_call(
        _b3_body,
        grid=(gr,),
        in_specs=[
            pl.BlockSpec((2, rb, 128), lambda i: (0, i, 0)),
            pl.BlockSpec((2, rb, 128), lambda i: (0, i, 0)),
            pl.BlockSpec((2, rb, 128), lambda i: (0, i, 0)),
            pl.BlockSpec((1, d_h2), lambda i: (0, 0)),
            pl.BlockSpec((d_h2, 16), lambda i: (0, 0)),
        ],
        out_specs=pl.BlockSpec((rb, 128), lambda i: (i, 0)),
        out_shape=jax.ShapeDtypeStruct((n_pad, 128), jnp.float32),
    )(agg2, deg_o, deg_i, b2r, w3p)

    agg3 = _make_agg_esplit(n, nch32)(h3n, src32, dst32, zeros128)

    out = pl.pallas_call(
        _b4_body,
        grid=(gr,),
        in_specs=[
            pl.BlockSpec((2, rb, 128), lambda i: (0, i, 0)),
            pl.BlockSpec((2, rb, 128), lambda i: (0, i, 0)),
            pl.BlockSpec((1, 16), lambda i: (0, 0)),
        ],
        out_specs=pl.BlockSpec((rb, d_out), lambda i: (i, 0)),
        out_shape=jax.ShapeDtypeStruct((n, d_out), jnp.float32),
    )(agg3, deg_i, b3p)

    return out


# speedup vs baseline: 2.8246x; 1.0402x over previous
import functools

import jax
import jax.numpy as jnp
from jax import lax
from jax.experimental import pallas as pl
from jax.experimental.pallas import tpu as pltpu
from jax.experimental.pallas import tpu_sc as plsc

_NC = 2
_NS = 16
_CH = 128


def _n_pad(n_nodes):
    return ((n_nodes + 128) // 128) * 128


def _make_degrees(n_nodes, n_chunks):
    n_pad = _n_pad(n_nodes)
    rpt = n_pad // _NS
    mesh = plsc.VectorSubcoreMesh(core_axis_name="c", subcore_axis_name="s")

    @functools.partial(
        pl.kernel,
        out_type=(jax.ShapeDtypeStruct((_NC, n_pad, 128), jnp.float32),
                  jax.ShapeDtypeStruct((_NC, n_pad, 128), jnp.float32)),
        mesh=mesh,
        scratch_types=[
            pltpu.VMEM((n_chunks, _CH), jnp.int32),
            pltpu.VMEM((n_chunks, _CH), jnp.int32),
            pltpu.VMEM((_CH, 128), jnp.float32),
            pltpu.VMEM_SHARED((n_pad, 128), jnp.float32),
        ],
    )
    def k(src2d, dst2d, ones_hbm, zeros_hbm, out_o, out_i,
          sidx, didx, ones_v, hist):
        c = lax.axis_index("c")
        s = lax.axis_index("s")
        w = c * _NS + s
        pltpu.sync_copy(ones_hbm, ones_v)
        pltpu.sync_copy(src2d.at[w], sidx)
        pltpu.sync_copy(dst2d.at[w], didx)

        for idx, out in ((sidx, out_o), (didx, out_i)):
            pltpu.sync_copy(zeros_hbm.at[pl.ds(s * rpt, rpt)],
                            hist.at[pl.ds(s * rpt, rpt)])
            plsc.subcore_barrier()

            def body(i, carry, idx=idx):
                pltpu.sync_copy(ones_v, hist.at[idx.at[i]], add=True)
                return carry

            lax.fori_loop(0, n_chunks, body, 0)
            plsc.subcore_barrier()
            pltpu.sync_copy(hist.at[pl.ds(s * rpt, rpt)],
                            out.at[c, pl.ds(s * rpt, rpt)])
            plsc.subcore_barrier()

    return k


_RCH = 64
_RBLK = 8
_RING = 4


def _agg_edge_loop_ring(table, agg, src4, dst4, w, sidx, didx, bufs,
                        gsems, ssems, sem_is, sem_id, n_blk):
    def gather(idx_row, slot):
        pltpu.async_copy(table.at[idx_row], bufs[slot], gsems[slot])

    def wait_gather(idx_row, slot):
        pltpu.make_async_copy(table.at[idx_row], bufs[slot],
                              gsems[slot]).wait()

    def scatter(idx_row, slot):
        pltpu.async_copy(bufs[slot], agg.at[idx_row], ssems[slot],
                         add=True)

    def wait_scatter(idx_row, slot):
        pltpu.make_async_copy(bufs[slot], agg.at[idx_row],
                              ssems[slot]).wait()

    pltpu.sync_copy(src4.at[w, 0], sidx.at[0])
    pltpu.sync_copy(dst4.at[w, 0], didx.at[0])
    gather(sidx.at[0, 0], 0)
    gather(sidx.at[0, 1], 1)

    def block(b, p):
        q = 1 - p
        for c in range(_RBLK):
            slot = c % _RING
            nslot = (c + 2) % _RING
            if c >= 2:
                wait_scatter(didx.at[p, c - 2], nslot)
            else:
                @pl.when(b > 0)
                def _():
                    wait_scatter(didx.at[p, c], nslot)
            if c == 2:
                @pl.when(b + 1 < n_blk)
                def _():
                    pltpu.async_copy(src4.at[w, b + 1], sidx.at[q], sem_is)
                    pltpu.async_copy(dst4.at[w, b + 1], didx.at[q], sem_id)
            if c < _RBLK - 2:
                gather(sidx.at[p, c + 2], nslot)
            else:
                @pl.when(b + 1 < n_blk)
                def _():
                    if c == _RBLK - 2:
                        pltpu.make_async_copy(src4.at[w, b + 1], sidx.at[q],
                                              sem_is).wait()
                        pltpu.make_async_copy(dst4.at[w, b + 1], didx.at[q],
                                              sem_id).wait()
                    gather(sidx.at[q, c - (_RBLK - 2)], nslot)
            wait_gather(sidx.at[p, c], slot)
            scatter(didx.at[p, c], slot)

    def body(j, carry):
        block(2 * j, 0)
        block(2 * j + 1, 1)
        return carry

    lax.fori_loop(0, n_blk // 2, body, 0)
    wait_scatter(didx.at[1, _RBLK - 2], (_RBLK - 2) % _RING)
    wait_scatter(didx.at[1, _RBLK - 1], (_RBLK - 1) % _RING)


def _make_agg_fsplit(n_nodes, d_half, n_blk):
    n_pad = _n_pad(n_nodes)
    rpt = n_pad // _NS
    mesh = plsc.VectorSubcoreMesh(core_axis_name="c", subcore_axis_name="s")

    @functools.partial(
        pl.kernel,
        out_type=jax.ShapeDtypeStruct((_NC, n_pad, d_half), jnp.float32),
        mesh=mesh,
        scratch_types=[
            pltpu.VMEM((2, _RBLK, _RCH), jnp.int32),
            pltpu.VMEM((2, _RBLK, _RCH), jnp.int32),
            [pltpu.VMEM((_RCH, d_half), jnp.float32)] * _RING,
            [pltpu.SemaphoreType.DMA] * _RING,
            [pltpu.SemaphoreType.DMA] * _RING,
            pltpu.VMEM_SHARED((n_pad, d_half), jnp.float32),
            pltpu.SemaphoreType.DMA,
            pltpu.SemaphoreType.DMA,
        ],
    )
    def k(h0, h1, src4, dst4, zeros_hbm, out,
          sidx, didx, bufs, gsems, ssems, agg, sem_is, sem_id):
        c = lax.axis_index("c")
        s = lax.axis_index("s")
        pltpu.sync_copy(zeros_hbm.at[pl.ds(s * rpt, rpt)],
                        agg.at[pl.ds(s * rpt, rpt)])
        plsc.subcore_barrier()

        @pl.when(c == 0)
        def _():
            _agg_edge_loop_ring(h0, agg, src4, dst4, s, sidx, didx, bufs,
                                gsems, ssems, sem_is, sem_id, n_blk)

        @pl.when(c == 1)
        def _():
            _agg_edge_loop_ring(h1, agg, src4, dst4, s, sidx, didx, bufs,
                                gsems, ssems, sem_is, sem_id, n_blk)

        plsc.subcore_barrier()
        pltpu.sync_copy(agg.at[pl.ds(s * rpt, rpt)],
                        out.at[c, pl.ds(s * rpt, rpt)])

    return k


def _make_agg_esplit(n_nodes, n_blk):
    n_pad = _n_pad(n_nodes)
    rpt = n_pad // _NS
    mesh = plsc.VectorSubcoreMesh(core_axis_name="c", subcore_axis_name="s")

    @functools.partial(
        pl.kernel,
        out_type=jax.ShapeDtypeStruct((_NC, n_pad, 128), jnp.float32),
        mesh=mesh,
        scratch_types=[
            pltpu.VMEM((2, _RBLK, _RCH), jnp.int32),
            pltpu.VMEM((2, _RBLK, _RCH), jnp.int32),
            [pltpu.VMEM((_RCH, 128), jnp.float32)] * _RING,
            [pltpu.SemaphoreType.DMA] * _RING,
            [pltpu.SemaphoreType.DMA] * _RING,
            pltpu.VMEM_SHARED((n_pad, 128), jnp.float32),
            pltpu.SemaphoreType.DMA,
            pltpu.SemaphoreType.DMA,
        ],
    )
    def k(h, src4, dst4, zeros_hbm, out,
          sidx, didx, bufs, gsems, ssems, agg, sem_is, sem_id):
        c = lax.axis_index("c")
        s = lax.axis_index("s")
        w = c * _NS + s
        pltpu.sync_copy(zeros_hbm.at[pl.ds(s * rpt, rpt)],
                        agg.at[pl.ds(s * rpt, rpt)])
        plsc.subcore_barrier()
        _agg_edge_loop_ring(h, agg, src4, dst4, w, sidx, didx, bufs,
                            gsems, ssems, sem_is, sem_id, n_blk)
        plsc.subcore_barrier()
        pltpu.sync_copy(agg.at[pl.ds(s * rpt, rpt)],
                        out.at[c, pl.ds(s * rpt, rpt)])

    return k


def _norm_from_deg(deg_ref):
    d = deg_ref[0, :, 0:1] + deg_ref[1, :, 0:1]
    return jnp.where(d > 0, lax.rsqrt(jnp.maximum(d, 1.0)), 0.0)


def _b1_body(x_ref, w_ref, dego_ref, out_ref):
    norm = _norm_from_deg(dego_ref)
    h = jnp.dot(x_ref[...], w_ref[...], preferred_element_type=jnp.float32)
    out_ref[0] = h * norm


def _b2_body(agg_ref, dego_ref, degi_ref, b1_ref, w2_ref, out_ref):
    norm_s = _norm_from_deg(dego_ref)
    norm_d = _norm_from_deg(degi_ref)
    x0 = jnp.maximum(agg_ref[0] * norm_d + b1_ref[:, 0:128], 0.0)
    x1 = jnp.maximum(agg_ref[1] * norm_d + b1_ref[:, 128:256], 0.0)
    h = (jnp.dot(x0, w2_ref[0:128, :], preferred_element_type=jnp.float32)
         + jnp.dot(x1, w2_ref[128:256, :], preferred_element_type=jnp.float32))
    hp = jnp.pad(h * norm_s, ((0, 0), (0, 128 - h.shape[1])))
    out_ref[...] = hp


def _b3_body(aggp_ref, dego_ref, degi_ref, b2_ref, w3_ref, out_ref):
    norm_s = _norm_from_deg(dego_ref)
    norm_d = _norm_from_deg(degi_ref)
    agg = aggp_ref[0, :, 0:32] + aggp_ref[1, :, 0:32]
    x = jnp.maximum(agg * norm_d + b2_ref[...], 0.0)
    h = jnp.dot(x, w3_ref[...], preferred_element_type=jnp.float32)
    hp = jnp.pad(h * norm_s, ((0, 0), (0, 128 - h.shape[1])))
    out_ref[...] = hp


def _b4_body(aggp_ref, degi_ref, b3_ref, out_ref):
    norm_d = _norm_from_deg(degi_ref)
    x = (aggp_ref[0, :, 0:16] + aggp_ref[1, :, 0:16]) * norm_d + b3_ref[...]
    cols = lax.broadcasted_iota(jnp.int32, x.shape, 1)
    valid = cols < 7
    xm = jnp.where(valid, x, -1e30)
    m = jnp.max(xm, axis=1, keepdims=True)
    ssum = jnp.sum(jnp.where(valid, jnp.exp(x - m), 0.0), axis=1,
                   keepdims=True)
    out_ref[...] = (x - m - jnp.log(ssum))[:, 0:7]


def kernel(features, edge_index, W1, b1, W2, b2, W3, b3):
    n, d_in = features.shape
    d_h1 = W1.shape[1]
    d_h2 = W2.shape[1]
    d_out = W3.shape[1]
    e = edge_index.shape[1]
    assert d_h1 == 256 and d_h2 == 32 and d_out == 7
    n_pad = _n_pad(n)

    epw = _NC * _NS * _RBLK * _RCH
    e_pad = ((e + epw - 1) // epw) * epw
    sent = jnp.full((e_pad - e,), n, dtype=jnp.int32)
    src = jnp.concatenate([edge_index[0].astype(jnp.int32), sent])
    dst = jnp.concatenate([edge_index[1].astype(jnp.int32), sent])
    nch32 = e_pad // (_NC * _NS * _CH)
    nblk16 = e_pad // (_NS * _RBLK * _RCH)
    nblk32 = e_pad // (_NC * _NS * _RBLK * _RCH)
    src32 = src.reshape(_NC * _NS, nblk32, _RBLK, _RCH)
    dst32 = dst.reshape(_NC * _NS, nblk32, _RBLK, _RCH)
    src16 = src.reshape(_NS, nblk16, _RBLK, _RCH)
    dst16 = dst.reshape(_NS, nblk16, _RBLK, _RCH)

    zeros128 = jnp.zeros((n_pad, 128), jnp.float32)
    b1r = b1.reshape(1, d_h1)
    b2r = b2.reshape(1, d_h2)
    w3p = jnp.pad(W3, ((0, 0), (0, 16 - d_out)))
    b3p = jnp.pad(b3, (0, 16 - d_out)).reshape(1, 16)

    srcd = src.reshape(_NC * _NS, nch32, _CH)
    dstd = dst.reshape(_NC * _NS, nch32, _CH)
    ones128 = jnp.ones((_CH, 128), jnp.float32)
    deg_o, deg_i = _make_degrees(n, nch32)(srcd, dstd, ones128, zeros128)

    rb = 1000
    gr = n // rb
    h1n = pl.pallas_call(
        _b1_body,
        grid=(gr, 2),
        in_specs=[
            pl.BlockSpec((rb, d_in), lambda i, j: (i, 0)),
            pl.BlockSpec((d_in, 128), lambda i, j: (0, j)),
            pl.BlockSpec((2, rb, 128), lambda i, j: (0, i, 0)),
        ],
        out_specs=pl.BlockSpec((1, rb, 128), lambda i, j: (j, i, 0)),
        out_shape=jax.ShapeDtypeStruct((2, n_pad, 128), jnp.float32),
    )(features, W1, deg_o)

    agg1 = _make_agg_fsplit(n, 128, nblk16)(
        h1n[0], h1n[1], src16, dst16, zeros128)

    h2n = pl.pallas_call(
        _b2_body,
        grid=(gr,),
        in_specs=[
            pl.BlockSpec((2, rb, 128), lambda i: (0, i, 0)),
            pl.BlockSpec((2, rb, 128), lambda i: (0, i, 0)),
            pl.BlockSpec((2, rb, 128), lambda i: (0, i, 0)),
            pl.BlockSpec((1, d_h1), lambda i: (0, 0)),
            pl.BlockSpec((d_h1, d_h2), lambda i: (0, 0)),
        ],
        out_specs=pl.BlockSpec((rb, 128), lambda i: (i, 0)),
        out_shape=jax.ShapeDtypeStruct((n_pad, 128), jnp.float32),
    )(agg1, deg_o, deg_i, b1r, W2)

    agg2 = _make_agg_esplit(n, nblk32)(h2n, src32, dst32, zeros128)

    h3n = pl.pallas_call(
        _b3_body,
        grid=(gr,),
        in_specs=[
            pl.BlockSpec((2, rb, 128), lambda i: (0, i, 0)),
            pl.BlockSpec((2, rb, 128), lambda i: (0, i, 0)),
            pl.BlockSpec((2, rb, 128), lambda i: (0, i, 0)),
            pl.BlockSpec((1, d_h2), lambda i: (0, 0)),
            pl.BlockSpec((d_h2, 16), lambda i: (0, 0)),
        ],
        out_specs=pl.BlockSpec((rb, 128), lambda i: (i, 0)),
        out_shape=jax.ShapeDtypeStruct((n_pad, 128), jnp.float32),
    )(agg2, deg_o, deg_i, b2r, w3p)

    agg3 = _make_agg_esplit(n, nblk32)(h3n, src32, dst32, zeros128)

    out = pl.pallas_call(
        _b4_body,
        grid=(gr,),
        in_specs=[
            pl.BlockSpec((2, rb, 128), lambda i: (0, i, 0)),
            pl.BlockSpec((2, rb, 128), lambda i: (0, i, 0)),
            pl.BlockSpec((1, 16), lambda i: (0, 0)),
        ],
        out_specs=pl.BlockSpec((rb, d_out), lambda i: (i, 0)),
        out_shape=jax.ShapeDtypeStruct((n, d_out), jnp.float32),
    )(agg3, deg_i, b3p)

    return out


# speedup vs baseline: 6.0759x; 2.1511x over previous
import functools

import jax
import jax.numpy as jnp
from jax import lax
from jax.experimental import pallas as pl
from jax.experimental.pallas import tpu as pltpu
from jax.experimental.pallas import tpu_sc as plsc

_NC = 2
_NS = 16
_CH = 128


def _n_pad(n_nodes):
    return ((n_nodes + 128) // 128) * 128


def _make_degrees(n_nodes, n_chunks):
    n_pad = _n_pad(n_nodes)
    rpt = n_pad // _NS
    mesh = plsc.VectorSubcoreMesh(core_axis_name="c", subcore_axis_name="s")

    @functools.partial(
        pl.kernel,
        out_type=(jax.ShapeDtypeStruct((_NC, n_pad, 128), jnp.float32),
                  jax.ShapeDtypeStruct((_NC, n_pad, 128), jnp.float32)),
        mesh=mesh,
        scratch_types=[
            pltpu.VMEM((n_chunks, _CH), jnp.int32),
            pltpu.VMEM((n_chunks, _CH), jnp.int32),
            pltpu.VMEM((_CH, 128), jnp.float32),
            pltpu.VMEM_SHARED((n_pad, 128), jnp.float32),
        ],
    )
    def k(src2d, dst2d, ones_hbm, zeros_hbm, out_o, out_i,
          sidx, didx, ones_v, hist):
        c = lax.axis_index("c")
        s = lax.axis_index("s")
        w = c * _NS + s
        pltpu.sync_copy(ones_hbm, ones_v)
        pltpu.sync_copy(src2d.at[w], sidx)
        pltpu.sync_copy(dst2d.at[w], didx)

        for idx, out in ((sidx, out_o), (didx, out_i)):
            pltpu.sync_copy(zeros_hbm.at[pl.ds(s * rpt, rpt)],
                            hist.at[pl.ds(s * rpt, rpt)])
            plsc.subcore_barrier()

            def body(i, carry, idx=idx):
                pltpu.sync_copy(ones_v, hist.at[idx.at[i]], add=True)
                return carry

            lax.fori_loop(0, n_chunks, body, 0)
            plsc.subcore_barrier()
            pltpu.sync_copy(hist.at[pl.ds(s * rpt, rpt)],
                            out.at[c, pl.ds(s * rpt, rpt)])
            plsc.subcore_barrier()

    return k


_RCH = 64
_RBLK = 8
_RING = 4


def _agg_edge_loop_ring(table, agg, src4, dst4, w, sidx, didx, bufs,
                        gsems, ssems, sem_is, sem_id, n_blk):
    def gather(idx_row, slot):
        pltpu.async_copy(table.at[idx_row], bufs[slot], gsems[slot])

    def wait_gather(idx_row, slot):
        pltpu.make_async_copy(table.at[idx_row], bufs[slot],
                              gsems[slot]).wait()

    def scatter(idx_row, slot):
        pltpu.async_copy(bufs[slot], agg.at[idx_row], ssems[slot],
                         add=True)

    def wait_scatter(idx_row, slot):
        pltpu.make_async_copy(bufs[slot], agg.at[idx_row],
                              ssems[slot]).wait()

    pltpu.sync_copy(src4.at[w, 0], sidx.at[0])
    pltpu.sync_copy(dst4.at[w, 0], didx.at[0])
    gather(sidx.at[0, 0], 0)
    gather(sidx.at[0, 1], 1)

    def block(b, p):
        q = 1 - p
        for c in range(_RBLK):
            slot = c % _RING
            nslot = (c + 2) % _RING
            if c >= 2:
                wait_scatter(didx.at[p, c - 2], nslot)
            else:
                @pl.when(b > 0)
                def _():
                    wait_scatter(didx.at[p, c], nslot)
            if c == 2:
                @pl.when(b + 1 < n_blk)
                def _():
                    pltpu.async_copy(src4.at[w, b + 1], sidx.at[q], sem_is)
                    pltpu.async_copy(dst4.at[w, b + 1], didx.at[q], sem_id)
            if c < _RBLK - 2:
                gather(sidx.at[p, c + 2], nslot)
            else:
                @pl.when(b + 1 < n_blk)
                def _():
                    if c == _RBLK - 2:
                        pltpu.make_async_copy(src4.at[w, b + 1], sidx.at[q],
                                              sem_is).wait()
                        pltpu.make_async_copy(dst4.at[w, b + 1], didx.at[q],
                                              sem_id).wait()
                    gather(sidx.at[q, c - (_RBLK - 2)], nslot)
            wait_gather(sidx.at[p, c], slot)
            scatter(didx.at[p, c], slot)

    def body(j, carry):
        block(2 * j, 0)
        block(2 * j + 1, 1)
        return carry

    lax.fori_loop(0, n_blk // 2, body, 0)
    wait_scatter(didx.at[1, _RBLK - 2], (_RBLK - 2) % _RING)
    wait_scatter(didx.at[1, _RBLK - 1], (_RBLK - 1) % _RING)


def _make_agg_fsplit(n_nodes, d_half, n_blk):
    n_pad = _n_pad(n_nodes)
    rpt = n_pad // _NS
    mesh = plsc.VectorSubcoreMesh(core_axis_name="c", subcore_axis_name="s")

    @functools.partial(
        pl.kernel,
        out_type=jax.ShapeDtypeStruct((_NC, n_pad, d_half), jnp.float32),
        mesh=mesh,
        scratch_types=[
            pltpu.VMEM((2, _RBLK, _RCH), jnp.int32),
            pltpu.VMEM((2, _RBLK, _RCH), jnp.int32),
            [pltpu.VMEM((_RCH, d_half), jnp.float32)] * _RING,
            [pltpu.SemaphoreType.DMA] * _RING,
            [pltpu.SemaphoreType.DMA] * _RING,
            pltpu.VMEM_SHARED((n_pad, d_half), jnp.float32),
            pltpu.SemaphoreType.DMA,
            pltpu.SemaphoreType.DMA,
        ],
    )
    def k(h0, h1, src4, dst4, zeros_hbm, out,
          sidx, didx, bufs, gsems, ssems, agg, sem_is, sem_id):
        c = lax.axis_index("c")
        s = lax.axis_index("s")
        pltpu.sync_copy(zeros_hbm.at[pl.ds(s * rpt, rpt)],
                        agg.at[pl.ds(s * rpt, rpt)])
        plsc.subcore_barrier()

        @pl.when(c == 0)
        def _():
            _agg_edge_loop_ring(h0, agg, src4, dst4, s, sidx, didx, bufs,
                                gsems, ssems, sem_is, sem_id, n_blk)

        @pl.when(c == 1)
        def _():
            _agg_edge_loop_ring(h1, agg, src4, dst4, s, sidx, didx, bufs,
                                gsems, ssems, sem_is, sem_id, n_blk)

        plsc.subcore_barrier()
        pltpu.sync_copy(agg.at[pl.ds(s * rpt, rpt)],
                        out.at[c, pl.ds(s * rpt, rpt)])

    return k


def _make_agg_esplit(n_nodes, n_blk):
    n_pad = _n_pad(n_nodes)
    rpt = n_pad // _NS
    mesh = plsc.VectorSubcoreMesh(core_axis_name="c", subcore_axis_name="s")

    @functools.partial(
        pl.kernel,
        out_type=jax.ShapeDtypeStruct((_NC, n_pad, 128), jnp.float32),
        mesh=mesh,
        scratch_types=[
            pltpu.VMEM((2, _RBLK, _RCH), jnp.int32),
            pltpu.VMEM((2, _RBLK, _RCH), jnp.int32),
            [pltpu.VMEM((_RCH, 128), jnp.float32)] * _RING,
            [pltpu.SemaphoreType.DMA] * _RING,
            [pltpu.SemaphoreType.DMA] * _RING,
            pltpu.VMEM_SHARED((n_pad, 128), jnp.float32),
            pltpu.SemaphoreType.DMA,
            pltpu.SemaphoreType.DMA,
        ],
    )
    def k(h, src4, dst4, zeros_hbm, out,
          sidx, didx, bufs, gsems, ssems, agg, sem_is, sem_id):
        c = lax.axis_index("c")
        s = lax.axis_index("s")
        w = c * _NS + s
        pltpu.sync_copy(zeros_hbm.at[pl.ds(s * rpt, rpt)],
                        agg.at[pl.ds(s * rpt, rpt)])
        plsc.subcore_barrier()
        _agg_edge_loop_ring(h, agg, src4, dst4, w, sidx, didx, bufs,
                            gsems, ssems, sem_is, sem_id, n_blk)
        plsc.subcore_barrier()
        pltpu.sync_copy(agg.at[pl.ds(s * rpt, rpt)],
                        out.at[c, pl.ds(s * rpt, rpt)])

    return k


def _norm_from_deg(deg_ref):
    d = deg_ref[0, :, 0:1] + deg_ref[1, :, 0:1]
    return jnp.where(d > 0, lax.rsqrt(jnp.maximum(d, 1.0)), 0.0)


def _b1_body(x_ref, w_ref, dego_ref, out_ref):
    norm = _norm_from_deg(dego_ref)
    h = jnp.dot(x_ref[...], w_ref[...], preferred_element_type=jnp.float32)
    out_ref[0] = h * norm


def _b2_body(agg_ref, dego_ref, degi_ref, b1_ref, w2_ref, out_ref):
    norm_s = _norm_from_deg(dego_ref)
    norm_d = _norm_from_deg(degi_ref)
    x0 = jnp.maximum(agg_ref[0] * norm_d + b1_ref[:, 0:128], 0.0)
    x1 = jnp.maximum(agg_ref[1] * norm_d + b1_ref[:, 128:256], 0.0)
    h = (jnp.dot(x0, w2_ref[0:128, :], preferred_element_type=jnp.float32)
         + jnp.dot(x1, w2_ref[128:256, :], preferred_element_type=jnp.float32))
    hp = jnp.pad(h * norm_s, ((0, 0), (0, 128 - h.shape[1])))
    out_ref[...] = hp


def _b3_body(aggp_ref, dego_ref, degi_ref, b2_ref, w3_ref, out_ref):
    norm_s = _norm_from_deg(dego_ref)
    norm_d = _norm_from_deg(degi_ref)
    agg = aggp_ref[0, :, 0:32] + aggp_ref[1, :, 0:32]
    x = jnp.maximum(agg * norm_d + b2_ref[...], 0.0)
    h = jnp.dot(x, w3_ref[...], preferred_element_type=jnp.float32)
    hp = jnp.pad(h * norm_s, ((0, 0), (0, 128 - h.shape[1])))
    out_ref[...] = hp


def _b4_body(aggp_ref, degi_ref, b3_ref, out_ref):
    norm_d = _norm_from_deg(degi_ref)
    x = (aggp_ref[0, :, 0:16] + aggp_ref[1, :, 0:16]) * norm_d + b3_ref[...]
    cols = lax.broadcasted_iota(jnp.int32, x.shape, 1)
    valid = cols < 7
    xm = jnp.where(valid, x, -1e30)
    m = jnp.max(xm, axis=1, keepdims=True)
    ssum = jnp.sum(jnp.where(valid, jnp.exp(x - m), 0.0), axis=1,
                   keepdims=True)
    out_ref[...] = (x - m - jnp.log(ssum))[:, 0:7]


def kernel(features, edge_index, W1, b1, W2, b2, W3, b3):
    n, d_in = features.shape
    d_h1 = W1.shape[1]
    d_h2 = W2.shape[1]
    d_out = W3.shape[1]
    e = edge_index.shape[1]
    assert d_h1 == 256 and d_h2 == 32 and d_out == 7
    n_pad = _n_pad(n)

    epw = _NC * _NS * _RBLK * _RCH
    e_pad = ((e + epw - 1) // epw) * epw
    sent = n + (jnp.arange(e_pad - e, dtype=jnp.int32) % (n_pad - n))
    src = jnp.concatenate([edge_index[0].astype(jnp.int32), sent])
    dst = jnp.concatenate([edge_index[1].astype(jnp.int32), sent])
    nch32 = e_pad // (_NC * _NS * _CH)
    nblk16 = e_pad // (_NS * _RBLK * _RCH)
    nblk32 = e_pad // (_NC * _NS * _RBLK * _RCH)
    src32 = src.reshape(_NC * _NS, nblk32, _RBLK, _RCH)
    dst32 = dst.reshape(_NC * _NS, nblk32, _RBLK, _RCH)
    src16 = src.reshape(_NS, nblk16, _RBLK, _RCH)
    dst16 = dst.reshape(_NS, nblk16, _RBLK, _RCH)

    zeros128 = jnp.zeros((n_pad, 128), jnp.float32)
    b1r = b1.reshape(1, d_h1)
    b2r = b2.reshape(1, d_h2)
    w3p = jnp.pad(W3, ((0, 0), (0, 16 - d_out)))
    b3p = jnp.pad(b3, (0, 16 - d_out)).reshape(1, 16)

    srcd = src.reshape(_NC * _NS, nch32, _CH)
    dstd = dst.reshape(_NC * _NS, nch32, _CH)
    ones128 = jnp.ones((_CH, 128), jnp.float32)
    deg_o, deg_i = _make_degrees(n, nch32)(srcd, dstd, ones128, zeros128)

    rb = 1000
    gr = n // rb
    h1n = pl.pallas_call(
        _b1_body,
        grid=(gr, 2),
        in_specs=[
            pl.BlockSpec((rb, d_in), lambda i, j: (i, 0)),
            pl.BlockSpec((d_in, 128), lambda i, j: (0, j)),
            pl.BlockSpec((2, rb, 128), lambda i, j: (0, i, 0)),
        ],
        out_specs=pl.BlockSpec((1, rb, 128), lambda i, j: (j, i, 0)),
        out_shape=jax.ShapeDtypeStruct((2, n_pad, 128), jnp.float32),
    )(features, W1, deg_o)

    agg1 = _make_agg_fsplit(n, 128, nblk16)(
        h1n[0], h1n[1], src16, dst16, zeros128)

    h2n = pl.pallas_call(
        _b2_body,
        grid=(gr,),
        in_specs=[
            pl.BlockSpec((2, rb, 128), lambda i: (0, i, 0)),
            pl.BlockSpec((2, rb, 128), lambda i: (0, i, 0)),
            pl.BlockSpec((2, rb, 128), lambda i: (0, i, 0)),
            pl.BlockSpec((1, d_h1), lambda i: (0, 0)),
            pl.BlockSpec((d_h1, d_h2), lambda i: (0, 0)),
        ],
        out_specs=pl.BlockSpec((rb, 128), lambda i: (i, 0)),
        out_shape=jax.ShapeDtypeStruct((n_pad, 128), jnp.float32),
    )(agg1, deg_o, deg_i, b1r, W2)

    agg2 = _make_agg_esplit(n, nblk32)(h2n, src32, dst32, zeros128)

    h3n = pl.pallas_call(
        _b3_body,
        grid=(gr,),
        in_specs=[
            pl.BlockSpec((2, rb, 128), lambda i: (0, i, 0)),
            pl.BlockSpec((2, rb, 128), lambda i: (0, i, 0)),
            pl.BlockSpec((2, rb, 128), lambda i: (0, i, 0)),
            pl.BlockSpec((1, d_h2), lambda i: (0, 0)),
            pl.BlockSpec((d_h2, 16), lambda i: (0, 0)),
        ],
        out_specs=pl.BlockSpec((rb, 128), lambda i: (i, 0)),
        out_shape=jax.ShapeDtypeStruct((n_pad, 128), jnp.float32),
    )(agg2, deg_o, deg_i, b2r, w3p)

    agg3 = _make_agg_esplit(n, nblk32)(h3n, src32, dst32, zeros128)

    out = pl.pallas_call(
        _b4_body,
        grid=(gr,),
        in_specs=[
            pl.BlockSpec((2, rb, 128), lambda i: (0, i, 0)),
            pl.BlockSpec((2, rb, 128), lambda i: (0, i, 0)),
            pl.BlockSpec((1, 16), lambda i: (0, 0)),
        ],
        out_specs=pl.BlockSpec((rb, d_out), lambda i: (i, 0)),
        out_shape=jax.ShapeDtypeStruct((n, d_out), jnp.float32),
    )(agg3, deg_i, b3p)

    return out


# speedup vs baseline: 6.2972x; 1.0364x over previous
import functools

import jax
import jax.numpy as jnp
from jax import lax
from jax.experimental import pallas as pl
from jax.experimental.pallas import tpu as pltpu
from jax.experimental.pallas import tpu_sc as plsc

_NC = 2
_NS = 16
_CH = 128


def _n_pad(n_nodes):
    return ((n_nodes + 128) // 128) * 128


def _make_degrees(n_nodes, n_chunks):
    n_pad = _n_pad(n_nodes)
    rpt = n_pad // _NS
    mesh = plsc.VectorSubcoreMesh(core_axis_name="c", subcore_axis_name="s")

    @functools.partial(
        pl.kernel,
        out_type=(jax.ShapeDtypeStruct((_NC, n_pad, 128), jnp.float32),
                  jax.ShapeDtypeStruct((_NC, n_pad, 128), jnp.float32)),
        mesh=mesh,
        scratch_types=[
            pltpu.VMEM((n_chunks, _CH), jnp.int32),
            pltpu.VMEM((n_chunks, _CH), jnp.int32),
            pltpu.VMEM((_CH, 128), jnp.float32),
            pltpu.VMEM_SHARED((n_pad, 128), jnp.float32),
            pltpu.SemaphoreType.DMA,
        ],
    )
    def k(src2d, dst2d, ones_hbm, zeros_hbm, out_o, out_i,
          sidx, didx, ones_v, hist, sem):
        c = lax.axis_index("c")
        s = lax.axis_index("s")
        w = c * _NS + s
        pltpu.sync_copy(ones_hbm, ones_v)
        pltpu.sync_copy(src2d.at[w], sidx)
        pltpu.sync_copy(dst2d.at[w], didx)

        for idx, out in ((sidx, out_o), (didx, out_i)):
            pltpu.sync_copy(zeros_hbm.at[pl.ds(s * rpt, rpt)],
                            hist.at[pl.ds(s * rpt, rpt)])
            plsc.subcore_barrier()

            def body(i, carry, idx=idx):
                for j in range(8):
                    pltpu.async_copy(ones_v, hist.at[idx.at[i * 8 + j]],
                                     sem, add=True)
                for j in range(8):
                    pltpu.make_async_copy(ones_v, hist.at[idx.at[i * 8 + j]],
                                          sem).wait()
                return carry

            lax.fori_loop(0, n_chunks // 8, body, 0)
            plsc.subcore_barrier()
            pltpu.sync_copy(hist.at[pl.ds(s * rpt, rpt)],
                            out.at[c, pl.ds(s * rpt, rpt)])
            plsc.subcore_barrier()

    return k


_RCH = 64
_RBLK = 8
_RING = 4


def _agg_edge_loop_ring(table, agg, src4, dst4, w, sidx, didx, bufs,
                        gsems, ssems, sem_is, sem_id, n_blk):
    def gather(idx_row, slot):
        pltpu.async_copy(table.at[idx_row], bufs[slot], gsems[slot])

    def wait_gather(idx_row, slot):
        pltpu.make_async_copy(table.at[idx_row], bufs[slot],
                              gsems[slot]).wait()

    def scatter(idx_row, slot):
        pltpu.async_copy(bufs[slot], agg.at[idx_row], ssems[slot],
                         add=True)

    def wait_scatter(idx_row, slot):
        pltpu.make_async_copy(bufs[slot], agg.at[idx_row],
                              ssems[slot]).wait()

    pltpu.sync_copy(src4.at[w, 0], sidx.at[0])
    pltpu.sync_copy(dst4.at[w, 0], didx.at[0])
    gather(sidx.at[0, 0], 0)
    gather(sidx.at[0, 1], 1)

    def block(b, p):
        q = 1 - p
        for c in range(_RBLK):
            slot = c % _RING
            nslot = (c + 2) % _RING
            if c >= 2:
                wait_scatter(didx.at[p, c - 2], nslot)
            else:
                @pl.when(b > 0)
                def _():
                    wait_scatter(didx.at[p, c], nslot)
            if c == 2:
                @pl.when(b + 1 < n_blk)
                def _():
                    pltpu.async_copy(src4.at[w, b + 1], sidx.at[q], sem_is)
                    pltpu.async_copy(dst4.at[w, b + 1], didx.at[q], sem_id)
            if c < _RBLK - 2:
                gather(sidx.at[p, c + 2], nslot)
            else:
                @pl.when(b + 1 < n_blk)
                def _():
                    if c == _RBLK - 2:
                        pltpu.make_async_copy(src4.at[w, b + 1], sidx.at[q],
                                              sem_is).wait()
                        pltpu.make_async_copy(dst4.at[w, b + 1], didx.at[q],
                                              sem_id).wait()
                    gather(sidx.at[q, c - (_RBLK - 2)], nslot)
            wait_gather(sidx.at[p, c], slot)
            scatter(didx.at[p, c], slot)

    def body(j, carry):
        block(2 * j, 0)
        block(2 * j + 1, 1)
        return carry

    lax.fori_loop(0, n_blk // 2, body, 0)
    wait_scatter(didx.at[1, _RBLK - 2], (_RBLK - 2) % _RING)
    wait_scatter(didx.at[1, _RBLK - 1], (_RBLK - 1) % _RING)


def _make_agg_fsplit(n_nodes, d_half, n_blk):
    n_pad = _n_pad(n_nodes)
    rpt = n_pad // _NS
    mesh = plsc.VectorSubcoreMesh(core_axis_name="c", subcore_axis_name="s")

    @functools.partial(
        pl.kernel,
        out_type=jax.ShapeDtypeStruct((_NC, n_pad, d_half), jnp.float32),
        mesh=mesh,
        scratch_types=[
            pltpu.VMEM((2, _RBLK, _RCH), jnp.int32),
            pltpu.VMEM((2, _RBLK, _RCH), jnp.int32),
            [pltpu.VMEM((_RCH, d_half), jnp.float32)] * _RING,
            [pltpu.SemaphoreType.DMA] * _RING,
            [pltpu.SemaphoreType.DMA] * _RING,
            pltpu.VMEM_SHARED((n_pad, d_half), jnp.float32),
            pltpu.SemaphoreType.DMA,
            pltpu.SemaphoreType.DMA,
        ],
    )
    def k(h0, h1, src4, dst4, zeros_hbm, out,
          sidx, didx, bufs, gsems, ssems, agg, sem_is, sem_id):
        c = lax.axis_index("c")
        s = lax.axis_index("s")
        pltpu.sync_copy(zeros_hbm.at[pl.ds(s * rpt, rpt)],
                        agg.at[pl.ds(s * rpt, rpt)])
        plsc.subcore_barrier()

        @pl.when(c == 0)
        def _():
            _agg_edge_loop_ring(h0, agg, src4, dst4, s, sidx, didx, bufs,
                                gsems, ssems, sem_is, sem_id, n_blk)

        @pl.when(c == 1)
        def _():
            _agg_edge_loop_ring(h1, agg, src4, dst4, s, sidx, didx, bufs,
                                gsems, ssems, sem_is, sem_id, n_blk)

        plsc.subcore_barrier()
        pltpu.sync_copy(agg.at[pl.ds(s * rpt, rpt)],
                        out.at[c, pl.ds(s * rpt, rpt)])

    return k


def _make_agg_esplit(n_nodes, n_blk):
    n_pad = _n_pad(n_nodes)
    rpt = n_pad // _NS
    mesh = plsc.VectorSubcoreMesh(core_axis_name="c", subcore_axis_name="s")

    @functools.partial(
        pl.kernel,
        out_type=jax.ShapeDtypeStruct((_NC, n_pad, 128), jnp.float32),
        mesh=mesh,
        scratch_types=[
            pltpu.VMEM((2, _RBLK, _RCH), jnp.int32),
            pltpu.VMEM((2, _RBLK, _RCH), jnp.int32),
            [pltpu.VMEM((_RCH, 128), jnp.float32)] * _RING,
            [pltpu.SemaphoreType.DMA] * _RING,
            [pltpu.SemaphoreType.DMA] * _RING,
            pltpu.VMEM_SHARED((n_pad, 128), jnp.float32),
            pltpu.SemaphoreType.DMA,
            pltpu.SemaphoreType.DMA,
        ],
    )
    def k(h, src4, dst4, zeros_hbm, out,
          sidx, didx, bufs, gsems, ssems, agg, sem_is, sem_id):
        c = lax.axis_index("c")
        s = lax.axis_index("s")
        w = c * _NS + s
        pltpu.sync_copy(zeros_hbm.at[pl.ds(s * rpt, rpt)],
                        agg.at[pl.ds(s * rpt, rpt)])
        plsc.subcore_barrier()
        _agg_edge_loop_ring(h, agg, src4, dst4, w, sidx, didx, bufs,
                            gsems, ssems, sem_is, sem_id, n_blk)
        plsc.subcore_barrier()
        pltpu.sync_copy(agg.at[pl.ds(s * rpt, rpt)],
                        out.at[c, pl.ds(s * rpt, rpt)])

    return k


def _norm_from_deg(deg_ref):
    d = deg_ref[0, :, 0:1] + deg_ref[1, :, 0:1]
    return jnp.where(d > 0, lax.rsqrt(jnp.maximum(d, 1.0)), 0.0)


def _b1a_body(x_ref, w_ref, out_ref):
    out_ref[0] = jnp.dot(x_ref[...], w_ref[...],
                         preferred_element_type=jnp.float32)


def _b1b_body(h_ref, dego_ref, out_ref):
    norm = _norm_from_deg(dego_ref)
    out_ref[0] = h_ref[0] * norm
    out_ref[1] = h_ref[1] * norm


def _b2_body(agg_ref, dego_ref, degi_ref, b1_ref, w2_ref, out_ref):
    norm_s = _norm_from_deg(dego_ref)
    norm_d = _norm_from_deg(degi_ref)
    x0 = jnp.maximum(agg_ref[0] * norm_d + b1_ref[:, 0:128], 0.0)
    x1 = jnp.maximum(agg_ref[1] * norm_d + b1_ref[:, 128:256], 0.0)
    h = (jnp.dot(x0, w2_ref[0:128, :], preferred_element_type=jnp.float32)
         + jnp.dot(x1, w2_ref[128:256, :], preferred_element_type=jnp.float32))
    hp = jnp.pad(h * norm_s, ((0, 0), (0, 128 - h.shape[1])))
    out_ref[...] = hp


def _b3_body(aggp_ref, dego_ref, degi_ref, b2_ref, w3_ref, out_ref):
    norm_s = _norm_from_deg(dego_ref)
    norm_d = _norm_from_deg(degi_ref)
    agg = aggp_ref[0, :, 0:32] + aggp_ref[1, :, 0:32]
    x = jnp.maximum(agg * norm_d + b2_ref[...], 0.0)
    h = jnp.dot(x, w3_ref[...], preferred_element_type=jnp.float32)
    hp = jnp.pad(h * norm_s, ((0, 0), (0, 128 - h.shape[1])))
    out_ref[...] = hp


def _b4_body(aggp_ref, degi_ref, b3_ref, out_ref):
    norm_d = _norm_from_deg(degi_ref)
    x = (aggp_ref[0, :, 0:16] + aggp_ref[1, :, 0:16]) * norm_d + b3_ref[...]
    cols = lax.broadcasted_iota(jnp.int32, x.shape, 1)
    valid = cols < 7
    xm = jnp.where(valid, x, -1e30)
    m = jnp.max(xm, axis=1, keepdims=True)
    ssum = jnp.sum(jnp.where(valid, jnp.exp(x - m), 0.0), axis=1,
                   keepdims=True)
    out_ref[...] = (x - m - jnp.log(ssum))[:, 0:7]


def kernel(features, edge_index, W1, b1, W2, b2, W3, b3):
    n, d_in = features.shape
    d_h1 = W1.shape[1]
    d_h2 = W2.shape[1]
    d_out = W3.shape[1]
    e = edge_index.shape[1]
    assert d_h1 == 256 and d_h2 == 32 and d_out == 7
    n_pad = _n_pad(n)

    epw = _NC * _NS * _RBLK * _RCH
    e_pad = ((e + epw - 1) // epw) * epw
    sent = n + (jnp.arange(e_pad - e, dtype=jnp.int32) % (n_pad - n))
    src = jnp.concatenate([edge_index[0].astype(jnp.int32), sent])
    dst = jnp.concatenate([edge_index[1].astype(jnp.int32), sent])
    nch32 = e_pad // (_NC * _NS * _CH)
    nblk16 = e_pad // (_NS * _RBLK * _RCH)
    nblk32 = e_pad // (_NC * _NS * _RBLK * _RCH)
    src32 = src.reshape(_NC * _NS, nblk32, _RBLK, _RCH)
    dst32 = dst.reshape(_NC * _NS, nblk32, _RBLK, _RCH)
    src16 = src.reshape(_NS, nblk16, _RBLK, _RCH)
    dst16 = dst.reshape(_NS, nblk16, _RBLK, _RCH)

    zeros128 = jnp.zeros((n_pad, 128), jnp.float32)
    b1r = b1.reshape(1, d_h1)
    b2r = b2.reshape(1, d_h2)
    w3p = jnp.pad(W3, ((0, 0), (0, 16 - d_out)))
    b3p = jnp.pad(b3, (0, 16 - d_out)).reshape(1, 16)

    srcd = src.reshape(_NC * _NS, nch32, _CH)
    dstd = dst.reshape(_NC * _NS, nch32, _CH)
    ones128 = jnp.ones((_CH, 128), jnp.float32)
    deg_o, deg_i = _make_degrees(n, nch32)(srcd, dstd, ones128, zeros128)

    rb = 1000
    gr = n // rb
    h1 = pl.pallas_call(
        _b1a_body,
        grid=(gr, 2),
        in_specs=[
            pl.BlockSpec((rb, d_in), lambda i, j: (i, 0)),
            pl.BlockSpec((d_in, 128), lambda i, j: (0, j)),
        ],
        out_specs=pl.BlockSpec((1, rb, 128), lambda i, j: (j, i, 0)),
        out_shape=jax.ShapeDtypeStruct((2, n_pad, 128), jnp.float32),
    )(features, W1)
    h1n = pl.pallas_call(
        _b1b_body,
        grid=(gr,),
        in_specs=[
            pl.BlockSpec((2, rb, 128), lambda i: (0, i, 0)),
            pl.BlockSpec((2, rb, 128), lambda i: (0, i, 0)),
        ],
        out_specs=pl.BlockSpec((2, rb, 128), lambda i: (0, i, 0)),
        out_shape=jax.ShapeDtypeStruct((2, n_pad, 128), jnp.float32),
    )(h1, deg_o)

    agg1 = _make_agg_fsplit(n, 128, nblk16)(
        h1n[0], h1n[1], src16, dst16, zeros128)

    h2n = pl.pallas_call(
        _b2_body,
        grid=(gr,),
        in_specs=[
            pl.BlockSpec((2, rb, 128), lambda i: (0, i, 0)),
            pl.BlockSpec((2, rb, 128), lambda i: (0, i, 0)),
            pl.BlockSpec((2, rb, 128), lambda i: (0, i, 0)),
            pl.BlockSpec((1, d_h1), lambda i: (0, 0)),
            pl.BlockSpec((d_h1, d_h2), lambda i: (0, 0)),
        ],
        out_specs=pl.BlockSpec((rb, 128), lambda i: (i, 0)),
        out_shape=jax.ShapeDtypeStruct((n_pad, 128), jnp.float32),
    )(agg1, deg_o, deg_i, b1r, W2)

    agg2 = _make_agg_esplit(n, nblk32)(h2n, src32, dst32, zeros128)

    h3n = pl.pallas_call(
        _b3_body,
        grid=(gr,),
        in_specs=[
            pl.BlockSpec((2, rb, 128), lambda i: (0, i, 0)),
            pl.BlockSpec((2, rb, 128), lambda i: (0, i, 0)),
            pl.BlockSpec((2, rb, 128), lambda i: (0, i, 0)),
            pl.BlockSpec((1, d_h2), lambda i: (0, 0)),
            pl.BlockSpec((d_h2, 16), lambda i: (0, 0)),
        ],
        out_specs=pl.BlockSpec((rb, 128), lambda i: (i, 0)),
        out_shape=jax.ShapeDtypeStruct((n_pad, 128), jnp.float32),
    )(agg2, deg_o, deg_i, b2r, w3p)

    agg3 = _make_agg_esplit(n, nblk32)(h3n, src32, dst32, zeros128)

    out = pl.pallas_call(
        _b4_body,
        grid=(gr,),
        in_specs=[
            pl.BlockSpec((2, rb, 128), lambda i: (0, i, 0)),
            pl.BlockSpec((2, rb, 128), lambda i: (0, i, 0)),
            pl.BlockSpec((1, 16), lambda i: (0, 0)),
        ],
        out_specs=pl.BlockSpec((rb, d_out), lambda i: (i, 0)),
        out_shape=jax.ShapeDtypeStruct((n, d_out), jnp.float32),
    )(agg3, deg_i, b3p)

    return out


# speedup vs baseline: 6.3044x; 1.0012x over previous
import functools

import jax
import jax.numpy as jnp
from jax import lax
from jax.experimental import pallas as pl
from jax.experimental.pallas import tpu as pltpu
from jax.experimental.pallas import tpu_sc as plsc

_NC = 2
_NS = 16
_CH = 128


def _n_pad(n_nodes):
    return ((n_nodes + 128) // 128) * 128


def _make_degrees(n_nodes, n_chunks):
    n_pad = _n_pad(n_nodes)
    rpt = n_pad // _NS
    mesh = plsc.VectorSubcoreMesh(core_axis_name="c", subcore_axis_name="s")

    @functools.partial(
        pl.kernel,
        out_type=(jax.ShapeDtypeStruct((_NC, n_pad, 128), jnp.float32),
                  jax.ShapeDtypeStruct((_NC, n_pad, 128), jnp.float32)),
        mesh=mesh,
        scratch_types=[
            pltpu.VMEM((n_chunks, _CH), jnp.int32),
            pltpu.VMEM((n_chunks, _CH), jnp.int32),
            pltpu.VMEM((_CH, 128), jnp.float32),
            pltpu.VMEM_SHARED((n_pad, 128), jnp.float32),
            pltpu.SemaphoreType.DMA,
        ],
    )
    def k(src2d, dst2d, ones_hbm, zeros_hbm, out_o, out_i,
          sidx, didx, ones_v, hist, sem):
        c = lax.axis_index("c")
        s = lax.axis_index("s")
        w = c * _NS + s
        pltpu.sync_copy(ones_hbm, ones_v)
        pltpu.sync_copy(src2d.at[w], sidx)
        pltpu.sync_copy(dst2d.at[w], didx)

        for idx, out in ((sidx, out_o), (didx, out_i)):
            pltpu.sync_copy(zeros_hbm.at[pl.ds(s * rpt, rpt)],
                            hist.at[pl.ds(s * rpt, rpt)])
            plsc.subcore_barrier()

            def body(i, carry, idx=idx):
                for j in range(8):
                    pltpu.async_copy(ones_v, hist.at[idx.at[i * 8 + j]],
                                     sem, add=True)
                for j in range(8):
                    pltpu.make_async_copy(ones_v, hist.at[idx.at[i * 8 + j]],
                                          sem).wait()
                return carry

            lax.fori_loop(0, n_chunks // 8, body, 0)
            plsc.subcore_barrier()
            pltpu.sync_copy(hist.at[pl.ds(s * rpt, rpt)],
                            out.at[c, pl.ds(s * rpt, rpt)])
            plsc.subcore_barrier()

    return k


_RCH = 64
_RBLK = 8
_RING = 4


def _agg_edge_loop_ring(table, agg, src4, dst4, w, sidx, didx, bufs,
                        gsems, ssems, sem_is, sem_id, n_blk):
    def gather(idx_row, slot):
        pltpu.async_copy(table.at[idx_row], bufs[slot], gsems[slot])

    def wait_gather(idx_row, slot):
        pltpu.make_async_copy(table.at[idx_row], bufs[slot],
                              gsems[slot]).wait()

    def scatter(idx_row, slot):
        pltpu.async_copy(bufs[slot], agg.at[idx_row], ssems[slot],
                         add=True)

    def wait_scatter(idx_row, slot):
        pltpu.make_async_copy(bufs[slot], agg.at[idx_row],
                              ssems[slot]).wait()

    pltpu.sync_copy(src4.at[w, 0], sidx.at[0])
    pltpu.sync_copy(dst4.at[w, 0], didx.at[0])
    gather(sidx.at[0, 0], 0)
    gather(sidx.at[0, 1], 1)

    def block(b, p):
        q = 1 - p
        for c in range(_RBLK):
            slot = c % _RING
            nslot = (c + 2) % _RING
            if c >= 2:
                wait_scatter(didx.at[p, c - 2], nslot)
            else:
                @pl.when(b > 0)
                def _():
                    wait_scatter(didx.at[p, c], nslot)
            if c == 2:
                @pl.when(b + 1 < n_blk)
                def _():
                    pltpu.async_copy(src4.at[w, b + 1], sidx.at[q], sem_is)
                    pltpu.async_copy(dst4.at[w, b + 1], didx.at[q], sem_id)
            if c < _RBLK - 2:
                gather(sidx.at[p, c + 2], nslot)
            else:
                @pl.when(b + 1 < n_blk)
                def _():
                    if c == _RBLK - 2:
                        pltpu.make_async_copy(src4.at[w, b + 1], sidx.at[q],
                                              sem_is).wait()
                        pltpu.make_async_copy(dst4.at[w, b + 1], didx.at[q],
                                              sem_id).wait()
                    gather(sidx.at[q, c - (_RBLK - 2)], nslot)
            wait_gather(sidx.at[p, c], slot)
            scatter(didx.at[p, c], slot)

    def body(j, carry):
        block(2 * j, 0)
        block(2 * j + 1, 1)
        return carry

    lax.fori_loop(0, n_blk // 2, body, 0)
    wait_scatter(didx.at[1, _RBLK - 2], (_RBLK - 2) % _RING)
    wait_scatter(didx.at[1, _RBLK - 1], (_RBLK - 1) % _RING)


def _make_agg_fsplit(n_nodes, d_half, n_blk):
    n_pad = _n_pad(n_nodes)
    rpt = n_pad // _NS
    mesh = plsc.VectorSubcoreMesh(core_axis_name="c", subcore_axis_name="s")

    @functools.partial(
        pl.kernel,
        out_type=jax.ShapeDtypeStruct((_NC, n_pad, d_half), jnp.float32),
        mesh=mesh,
        scratch_types=[
            pltpu.VMEM((2, _RBLK, _RCH), jnp.int32),
            pltpu.VMEM((2, _RBLK, _RCH), jnp.int32),
            [pltpu.VMEM((_RCH, d_half), jnp.float32)] * _RING,
            [pltpu.SemaphoreType.DMA] * _RING,
            [pltpu.SemaphoreType.DMA] * _RING,
            pltpu.VMEM_SHARED((n_pad, d_half), jnp.float32),
            pltpu.SemaphoreType.DMA,
            pltpu.SemaphoreType.DMA,
        ],
    )
    def k(h0, h1, src4, dst4, zeros_hbm, out,
          sidx, didx, bufs, gsems, ssems, agg, sem_is, sem_id):
        c = lax.axis_index("c")
        s = lax.axis_index("s")
        pltpu.sync_copy(zeros_hbm.at[pl.ds(s * rpt, rpt)],
                        agg.at[pl.ds(s * rpt, rpt)])
        plsc.subcore_barrier()

        @pl.when(c == 0)
        def _():
            _agg_edge_loop_ring(h0, agg, src4, dst4, s, sidx, didx, bufs,
                                gsems, ssems, sem_is, sem_id, n_blk)

        @pl.when(c == 1)
        def _():
            _agg_edge_loop_ring(h1, agg, src4, dst4, s, sidx, didx, bufs,
                                gsems, ssems, sem_is, sem_id, n_blk)

        plsc.subcore_barrier()
        pltpu.sync_copy(agg.at[pl.ds(s * rpt, rpt)],
                        out.at[c, pl.ds(s * rpt, rpt)])

    return k


def _make_agg_esplit(n_nodes, n_blk):
    n_pad = _n_pad(n_nodes)
    rpt = n_pad // _NS
    mesh = plsc.VectorSubcoreMesh(core_axis_name="c", subcore_axis_name="s")

    @functools.partial(
        pl.kernel,
        out_type=jax.ShapeDtypeStruct((_NC, n_pad, 128), jnp.float32),
        mesh=mesh,
        scratch_types=[
            pltpu.VMEM((2, _RBLK, _RCH), jnp.int32),
            pltpu.VMEM((2, _RBLK, _RCH), jnp.int32),
            [pltpu.VMEM((_RCH, 128), jnp.float32)] * _RING,
            [pltpu.SemaphoreType.DMA] * _RING,
            [pltpu.SemaphoreType.DMA] * _RING,
            pltpu.VMEM_SHARED((n_pad, 128), jnp.float32),
            pltpu.SemaphoreType.DMA,
            pltpu.SemaphoreType.DMA,
        ],
    )
    def k(h, src4, dst4, zeros_hbm, out,
          sidx, didx, bufs, gsems, ssems, agg, sem_is, sem_id):
        c = lax.axis_index("c")
        s = lax.axis_index("s")
        w = c * _NS + s
        pltpu.sync_copy(zeros_hbm.at[pl.ds(s * rpt, rpt)],
                        agg.at[pl.ds(s * rpt, rpt)])
        plsc.subcore_barrier()
        _agg_edge_loop_ring(h, agg, src4, dst4, w, sidx, didx, bufs,
                            gsems, ssems, sem_is, sem_id, n_blk)
        plsc.subcore_barrier()
        pltpu.sync_copy(agg.at[pl.ds(s * rpt, rpt)],
                        out.at[c, pl.ds(s * rpt, rpt)])

    return k


def _norm_from_deg(deg_ref):
    d = deg_ref[0, :, 0:1] + deg_ref[1, :, 0:1]
    return jnp.where(d > 0, lax.rsqrt(jnp.maximum(d, 1.0)), 0.0)


def _b1a_body(x_ref, w_ref, out_ref):
    out_ref[0] = jnp.dot(x_ref[...].astype(jnp.bfloat16),
                         w_ref[...].astype(jnp.bfloat16),
                         preferred_element_type=jnp.float32)


def _b1b_body(h_ref, dego_ref, out_ref):
    norm = _norm_from_deg(dego_ref)
    out_ref[0] = h_ref[0] * norm
    out_ref[1] = h_ref[1] * norm


def _b2_body(agg_ref, dego_ref, degi_ref, b1_ref, w2_ref, out_ref):
    norm_s = _norm_from_deg(dego_ref)
    norm_d = _norm_from_deg(degi_ref)
    x0 = jnp.maximum(agg_ref[0] * norm_d + b1_ref[:, 0:128], 0.0)
    x1 = jnp.maximum(agg_ref[1] * norm_d + b1_ref[:, 128:256], 0.0)
    h = (jnp.dot(x0, w2_ref[0:128, :], preferred_element_type=jnp.float32)
         + jnp.dot(x1, w2_ref[128:256, :], preferred_element_type=jnp.float32))
    hp = jnp.pad(h * norm_s, ((0, 0), (0, 128 - h.shape[1])))
    out_ref[...] = hp


def _b3_body(aggp_ref, dego_ref, degi_ref, b2_ref, w3_ref, out_ref):
    norm_s = _norm_from_deg(dego_ref)
    norm_d = _norm_from_deg(degi_ref)
    agg = aggp_ref[0, :, 0:32] + aggp_ref[1, :, 0:32]
    x = jnp.maximum(agg * norm_d + b2_ref[...], 0.0)
    h = jnp.dot(x, w3_ref[...], preferred_element_type=jnp.float32)
    hp = jnp.pad(h * norm_s, ((0, 0), (0, 128 - h.shape[1])))
    out_ref[...] = hp


def _b4_body(aggp_ref, degi_ref, b3_ref, out_ref):
    norm_d = _norm_from_deg(degi_ref)
    x = (aggp_ref[0, :, 0:16] + aggp_ref[1, :, 0:16]) * norm_d + b3_ref[...]
    cols = lax.broadcasted_iota(jnp.int32, x.shape, 1)
    valid = cols < 7
    xm = jnp.where(valid, x, -1e30)
    m = jnp.max(xm, axis=1, keepdims=True)
    ssum = jnp.sum(jnp.where(valid, jnp.exp(x - m), 0.0), axis=1,
                   keepdims=True)
    out_ref[...] = (x - m - jnp.log(ssum))[:, 0:7]


def kernel(features, edge_index, W1, b1, W2, b2, W3, b3):
    n, d_in = features.shape
    d_h1 = W1.shape[1]
    d_h2 = W2.shape[1]
    d_out = W3.shape[1]
    e = edge_index.shape[1]
    assert d_h1 == 256 and d_h2 == 32 and d_out == 7
    n_pad = _n_pad(n)

    epw = _NC * _NS * _RBLK * _RCH
    e_pad = ((e + epw - 1) // epw) * epw
    sent = n + (jnp.arange(e_pad - e, dtype=jnp.int32) % (n_pad - n))
    src = jnp.concatenate([edge_index[0].astype(jnp.int32), sent])
    dst = jnp.concatenate([edge_index[1].astype(jnp.int32), sent])
    nch32 = e_pad // (_NC * _NS * _CH)
    nblk16 = e_pad // (_NS * _RBLK * _RCH)
    nblk32 = e_pad // (_NC * _NS * _RBLK * _RCH)
    src32 = src.reshape(_NC * _NS, nblk32, _RBLK, _RCH)
    dst32 = dst.reshape(_NC * _NS, nblk32, _RBLK, _RCH)
    src16 = src.reshape(_NS, nblk16, _RBLK, _RCH)
    dst16 = dst.reshape(_NS, nblk16, _RBLK, _RCH)

    zeros128 = jnp.zeros((n_pad, 128), jnp.float32)
    b1r = b1.reshape(1, d_h1)
    b2r = b2.reshape(1, d_h2)
    w3p = jnp.pad(W3, ((0, 0), (0, 16 - d_out)))
    b3p = jnp.pad(b3, (0, 16 - d_out)).reshape(1, 16)

    srcd = src.reshape(_NC * _NS, nch32, _CH)
    dstd = dst.reshape(_NC * _NS, nch32, _CH)
    ones128 = jnp.ones((_CH, 128), jnp.float32)
    deg_o, deg_i = _make_degrees(n, nch32)(srcd, dstd, ones128, zeros128)

    rb = 1000
    gr = n // rb
    h1 = pl.pallas_call(
        _b1a_body,
        grid=(gr, 2),
        in_specs=[
            pl.BlockSpec((rb, d_in), lambda i, j: (i, 0)),
            pl.BlockSpec((d_in, 128), lambda i, j: (0, j)),
        ],
        out_specs=pl.BlockSpec((1, rb, 128), lambda i, j: (j, i, 0)),
        out_shape=jax.ShapeDtypeStruct((2, n_pad, 128), jnp.float32),
    )(features, W1)
    h1n = pl.pallas_call(
        _b1b_body,
        grid=(gr,),
        in_specs=[
            pl.BlockSpec((2, rb, 128), lambda i: (0, i, 0)),
            pl.BlockSpec((2, rb, 128), lambda i: (0, i, 0)),
        ],
        out_specs=pl.BlockSpec((2, rb, 128), lambda i: (0, i, 0)),
        out_shape=jax.ShapeDtypeStruct((2, n_pad, 128), jnp.float32),
    )(h1, deg_o)

    agg1 = _make_agg_fsplit(n, 128, nblk16)(
        h1n[0], h1n[1], src16, dst16, zeros128)

    h2n = pl.pallas_call(
        _b2_body,
        grid=(gr,),
        in_specs=[
            pl.BlockSpec((2, rb, 128), lambda i: (0, i, 0)),
            pl.BlockSpec((2, rb, 128), lambda i: (0, i, 0)),
            pl.BlockSpec((2, rb, 128), lambda i: (0, i, 0)),
            pl.BlockSpec((1, d_h1), lambda i: (0, 0)),
            pl.BlockSpec((d_h1, d_h2), lambda i: (0, 0)),
        ],
        out_specs=pl.BlockSpec((rb, 128), lambda i: (i, 0)),
        out_shape=jax.ShapeDtypeStruct((n_pad, 128), jnp.float32),
    )(agg1, deg_o, deg_i, b1r, W2)

    agg2 = _make_agg_esplit(n, nblk32)(h2n, src32, dst32, zeros128)

    h3n = pl.pallas_call(
        _b3_body,
        grid=(gr,),
        in_specs=[
            pl.BlockSpec((2, rb, 128), lambda i: (0, i, 0)),
            pl.BlockSpec((2, rb, 128), lambda i: (0, i, 0)),
            pl.BlockSpec((2, rb, 128), lambda i: (0, i, 0)),
            pl.BlockSpec((1, d_h2), lambda i: (0, 0)),
            pl.BlockSpec((d_h2, 16), lambda i: (0, 0)),
        ],
        out_specs=pl.BlockSpec((rb, 128), lambda i: (i, 0)),
        out_shape=jax.ShapeDtypeStruct((n_pad, 128), jnp.float32),
    )(agg2, deg_o, deg_i, b2r, w3p)

    agg3 = _make_agg_esplit(n, nblk32)(h3n, src32, dst32, zeros128)

    out = pl.pallas_call(
        _b4_body,
        grid=(gr,),
        in_specs=[
            pl.BlockSpec((2, rb, 128), lambda i: (0, i, 0)),
            pl.BlockSpec((2, rb, 128), lambda i: (0, i, 0)),
            pl.BlockSpec((1, 16), lambda i: (0, 0)),
        ],
        out_specs=pl.BlockSpec((rb, d_out), lambda i: (i, 0)),
        out_shape=jax.ShapeDtypeStruct((n, d_out), jnp.float32),
    )(agg3, deg_i, b3p)

    return out
